# baseline (speedup 1.0000x reference)
import jax
import jax.numpy as jnp
from jax import lax
from jax.experimental import pallas as pl
from jax.experimental.pallas import tpu as pltpu

N = 100000
T = 64
B = 64
D = 128
BN = 1000
G = N // BN


def _body(len_ref, to_ref, po_ref, th_ref, ph_ref, hid_ref, ba_ref,
          out_ref, hint_ref, hidn_ref):
    i = pl.program_id(0)

    zero = jnp.zeros((1, 1), jnp.float32)

    @pl.when(i == 0)
    def _init():
        out_ref[...] = zero
        hint_ref[...] = zero
        hidn_ref[...] = zero

    to = to_ref[0]
    po = po_ref[0]
    d0 = po - to
    s_out = jnp.sum(d0 * d0, keepdims=True)

    ba = ba_ref[...]
    lenr = len_ref[...]
    b_iota = lax.broadcasted_iota(jnp.int32, (BN, B), 1)
    lmat = jnp.where(ba == b_iota, jnp.broadcast_to(lenr, (BN, B)), 0)
    thr = jnp.sum(lmat, axis=1, keepdims=True)
    t_iota = lax.broadcasted_iota(jnp.int32, (BN, T), 1)
    mask = (t_iota < thr).astype(jnp.float32)
    dh = ph_ref[...] - th_ref[...]
    s_hint = jnp.sum(dh * dh * mask, keepdims=True).reshape(1, 1)

    h = hid_ref[...]
    rs = jnp.sum(h * h, axis=1, keepdims=True)
    s_hid = jnp.sum(jnp.sqrt(rs), axis=0, keepdims=True)

    out_ref[...] += s_out
    hint_ref[...] += s_hint
    hidn_ref[...] += s_hid


def kernel(truth_out, pred_out, truth_hint, pred_hint, hidden,
           edge_index, batch_assign, length):
    del edge_index
    to2 = truth_out.reshape(G, 1, BN)
    po2 = pred_out.reshape(G, 1, BN)
    ba2 = batch_assign.reshape(N, 1)
    len2 = length.reshape(1, B)

    scal = pl.BlockSpec((1, 1), lambda i: (0, 0))
    out, hint, hid = pl.pallas_call(
        _body,
        grid=(G,),
        in_specs=[
            pl.BlockSpec((1, B), lambda i: (0, 0)),
            pl.BlockSpec((1, 1, BN), lambda i: (i, 0, 0)),
            pl.BlockSpec((1, 1, BN), lambda i: (i, 0, 0)),
            pl.BlockSpec((BN, T), lambda i: (i, 0)),
            pl.BlockSpec((BN, T), lambda i: (i, 0)),
            pl.BlockSpec((BN, D), lambda i: (i, 0)),
            pl.BlockSpec((BN, 1), lambda i: (i, 0)),
        ],
        out_specs=[scal, scal, scal],
        out_shape=[jax.ShapeDtypeStruct((1, 1), jnp.float32)] * 3,
        compiler_params=pltpu.CompilerParams(
            dimension_semantics=("arbitrary",),
        ),
    )(len2, to2, po2, truth_hint, pred_hint, hidden, ba2)

    output_loss = (out[:, 0] / N).astype(jnp.float32)
    hint_loss = (hint[:, 0] / (N * T)).astype(jnp.float32)
    hidden_loss = (hid[0, 0] / N).astype(jnp.float32)
    return (output_loss, hint_loss, hidden_loss)
